# baseline (speedup 1.0000x reference)
import jax
import jax.numpy as jnp
from jax import lax
from jax.experimental import pallas as pl
from jax.experimental.pallas import tpu as pltpu
from jax.experimental.pallas import tpu_sc as plsc

_NUM_EMBS = 1000000
_EMSIZE = 64
_T, _B = 512, 1024
_N = _T * _B

_NC = 2
_NS = 16
_NW = _NC * _NS
_K = 4
_TPC = _T // _K
_RPC = _TPC // _NW
_C = 512
_NBUF = 2


def _make_body(t_off):
    def _body(idx_hbm, emb_hbm, out_hbm, idx_v,
              rows0, rows1, gsem0, gsem1, osem0, osem1):
        wid = lax.axis_index("s") * _NC + lax.axis_index("c")
        rows = (rows0, rows1)
        gsem = (gsem0, gsem1)
        osem = (osem0, osem1)

        pltpu.sync_copy(idx_hbm.at[pl.ds(t_off + wid * _RPC, _RPC)], idx_v)

        def gather(g, b):
            return pltpu.make_async_copy(
                emb_hbm.at[idx_v.at[g, pl.ds(b * _C, _C)]], rows[b], gsem[b])

        def out_copy(g, b):
            return pltpu.make_async_copy(
                rows[b],
                out_hbm.at[wid * _RPC + g,
                           pl.ds(b * _C, _C), pl.ds(0, _EMSIZE)],
                osem[b])

        gather(0, 0).start()
        gather(0, 1).start()

        def group(g, carry):
            for b in range(_NBUF):
                gather(g, b).wait()
                od = out_copy(g, b)
                od.start()
                od.wait()

                @pl.when(g + 1 < _RPC)
                def _():
                    gather(g + 1, b).start()
            return carry

        lax.fori_loop(0, _RPC, group, 0)

    return _body


def _run_slab(t_off, idx, embedding):
    mesh = plsc.VectorSubcoreMesh(core_axis_name="c", subcore_axis_name="s")
    return pl.kernel(
        _make_body(t_off),
        mesh=mesh,
        compiler_params=pltpu.CompilerParams(use_tc_tiling_on_sc=False),
        out_type=jax.ShapeDtypeStruct((_TPC, _B, 2 * _EMSIZE), jnp.float32),
        scratch_types=[
            pltpu.VMEM((_RPC, _B), jnp.int32),
            pltpu.VMEM((_C, _EMSIZE), jnp.float32),
            pltpu.VMEM((_C, _EMSIZE), jnp.float32),
            pltpu.SemaphoreType.DMA,
            pltpu.SemaphoreType.DMA,
            pltpu.SemaphoreType.DMA,
            pltpu.SemaphoreType.DMA,
        ],
    )(idx, embedding)


def kernel(x, embedding, single_eval_pos):
    xs = x[..., 0]
    bad = jnp.isnan(xs) | jnp.isinf(xs)
    idx = jnp.clip(xs, 0.0, float(_NUM_EMBS - 2)).astype(jnp.int32)
    last_row = embedding[_NUM_EMBS - 1]
    outs = []
    for k in range(_K):
        part = _run_slab(k * _TPC, idx, embedding)
        bad_k = bad[k * _TPC:(k + 1) * _TPC]
        outs.append(jnp.where(bad_k[:, :, None],
                              last_row[None, None, :],
                              part[..., :_EMSIZE]))
    return jnp.concatenate(outs, axis=0)
